# baseline (speedup 1.0000x reference)
import jax
import jax.numpy as jnp
from jax import lax
from jax.experimental import pallas as pl
from jax.experimental.pallas import tpu as pltpu
from jax.experimental.pallas import tpu_sc as plsc

NC = 2
NS = 16
L = 16
NW = NC * NS

B = 16384
D = 64
BPW = B // NW
CHUNK = 128
NCH = BPW // CHUNK
HALF = 256

_PARAMS = pltpu.CompilerParams(needs_layout_passes=False,
                               use_tc_tiling_on_sc=True)


def _fire_rows(idx, table, dst, sem, p):
    def fire(g, _):
        gg = p * (HALF // L) + g
        tv = idx[gg // (CHUNK // L), pl.ds((gg % (CHUNK // L)) * L, L)]
        for j in range(L):
            t = tv[j]
            pltpu.make_async_copy(
                table.at[lax.shift_right_logical(t, 3), jnp.bitwise_and(t, 7)],
                dst.at[g * L + j], sem).start()
        return 0
    lax.fori_loop(0, HALF // L, fire, 0)


def _drain_rows(table, dst, sem):
    def drain(lr, _):
        pltpu.make_async_copy(table.at[0, 0], dst.at[lr], sem).wait()
        return 0
    lax.fori_loop(0, HALF, drain, 0)


def _body_a(u3, v3, ue3, ub_hbm, ib_hbm, uu_hbm, bs_hbm,
            idx_u, idx_v, U0, U1, bu, bv, sem, semb):
    wid = lax.axis_index("s") * NC + lax.axis_index("c")
    base = wid * BPW

    pltpu.sync_copy(u3.at[wid], idx_u)
    pltpu.sync_copy(v3.at[wid], idx_v)

    bias_cps = []
    for k in range(NCH):
        sl = pl.ds(k * CHUNK, CHUNK)
        bias_cps.append(pltpu.async_copy(ub_hbm.at[idx_u.at[k]], bu.at[sl], semb))
        bias_cps.append(pltpu.async_copy(ib_hbm.at[idx_v.at[k]], bv.at[sl], semb))

    _fire_rows(idx_u, ue3, U0, sem, 0)
    _fire_rows(idx_u, ue3, U1, sem, 1)
    for c in bias_cps:
        c.wait()

    for g in range(BPW // L):
        sl = pl.ds(g * L, L)
        bu[sl] = bu[sl] + bv[sl]
    pltpu.sync_copy(bu, bs_hbm.at[pl.ds(base, BPW)])

    _drain_rows(ue3, U0, sem)
    pltpu.sync_copy(U0, uu_hbm.at[pl.ds(base, HALF)])
    _drain_rows(ue3, U1, sem)
    pltpu.sync_copy(U1, uu_hbm.at[pl.ds(base + HALF, HALF)])


def _body_b(v3, ie3, uu_hbm, bs_hbm, out_hbm,
            idx_v, V0, V1, U_, bs, outb, sem, semu):
    wid = lax.axis_index("s") * NC + lax.axis_index("c")
    base = wid * BPW

    pltpu.sync_copy(v3.at[wid], idx_v)
    _fire_rows(idx_v, ie3, V0, sem, 0)
    _fire_rows(idx_v, ie3, V1, sem, 1)
    pltpu.sync_copy(bs_hbm.at[pl.ds(base, BPW)], bs)

    lane = lax.iota(jnp.int32, L)

    for p in range(2):
        V_ = (V0, V1)[p]
        cpu = pltpu.make_async_copy(
            uu_hbm.at[pl.ds(base + p * HALF, HALF)], U_, semu)
        cpu.start()
        _drain_rows(ie3, V_, sem)
        cpu.wait()

        def group(g, _, V_=V_, p=p):
            grp = jnp.zeros((L,), jnp.float32)
            for r_off in range(L):
                lr = g * L + r_off
                acc = U_[lr, pl.ds(0, L)] * V_[lr, pl.ds(0, L)]
                for c in range(1, D // L):
                    acc = acc + U_[lr, pl.ds(c * L, L)] * V_[lr, pl.ds(c * L, L)]
                grp = jnp.where(lane == r_off, jnp.sum(acc), grp)
            sl = pl.ds(p * HALF + g * L, L)
            x = grp + bs[sl]
            outb[sl] = 1.0 / (1.0 + jnp.exp(-x))
            return 0

        lax.fori_loop(0, HALF // L, group, 0)

    pltpu.sync_copy(outb, out_hbm.at[pl.ds(base, BPW)])


@jax.jit
def _mf_sc(u3, v3, ue3, ie3, ub, ib):
    mesh = plsc.VectorSubcoreMesh(core_axis_name="c", subcore_axis_name="s",
                                  num_cores=NC, num_subcores=NS)
    uu, bs = pl.kernel(
        _body_a,
        out_type=(jax.ShapeDtypeStruct((B, D), jnp.float32),
                  jax.ShapeDtypeStruct((B,), jnp.float32)),
        mesh=mesh,
        compiler_params=_PARAMS,
        scratch_types=[
            pltpu.VMEM((NCH, CHUNK), jnp.int32),
            pltpu.VMEM((NCH, CHUNK), jnp.int32),
            pltpu.VMEM((HALF, D), jnp.float32),
            pltpu.VMEM((HALF, D), jnp.float32),
            pltpu.VMEM((BPW,), jnp.float32),
            pltpu.VMEM((BPW,), jnp.float32),
            pltpu.SemaphoreType.DMA,
            pltpu.SemaphoreType.DMA,
        ],
    )(u3, v3, ue3, ub, ib)
    return pl.kernel(
        _body_b,
        out_type=jax.ShapeDtypeStruct((B,), jnp.float32),
        mesh=mesh,
        compiler_params=_PARAMS,
        scratch_types=[
            pltpu.VMEM((NCH, CHUNK), jnp.int32),
            pltpu.VMEM((HALF, D), jnp.float32),
            pltpu.VMEM((HALF, D), jnp.float32),
            pltpu.VMEM((HALF, D), jnp.float32),
            pltpu.VMEM((BPW,), jnp.float32),
            pltpu.VMEM((BPW,), jnp.float32),
            pltpu.SemaphoreType.DMA,
            pltpu.SemaphoreType.DMA,
        ],
    )(v3, ie3, uu, bs)


def kernel(u, v, user_emb, item_emb, user_bias, item_bias):
    u3 = u.astype(jnp.int32).reshape(NW, NCH, CHUNK)
    v3 = v.astype(jnp.int32).reshape(NW, NCH, CHUNK)
    ue3 = user_emb.reshape(12500, 8, D)
    ie3 = item_emb.reshape(12500, 8, D)
    ub = user_bias.reshape(-1)
    ib = item_bias.reshape(-1)
    return _mf_sc(u3, v3, ue3, ie3, ub, ib)


# speedup vs baseline: 1.1703x; 1.1703x over previous
import jax
import jax.numpy as jnp
from jax import lax
from jax.experimental import pallas as pl
from jax.experimental.pallas import tpu as pltpu
from jax.experimental.pallas import tpu_sc as plsc

NC = 2
NS = 16
L = 16
NW = NC * NS

B = 16384
D = 64
BPW = B // NW
CHUNK = 128
NCH = BPW // CHUNK
HALF = 256


def _body(u3, v3, ue3, ie3, ub_hbm, ib_hbm, out_hbm,
          idx_u, idx_v, U, V, bu, bv, outb, sem, semb):
    wid = lax.axis_index("s") * NC + lax.axis_index("c")
    base = wid * BPW

    pltpu.sync_copy(u3.at[wid], idx_u)
    pltpu.sync_copy(v3.at[wid], idx_v)

    bias_cps = []
    for k in range(NCH):
        sl = pl.ds(k * CHUNK, CHUNK)
        bias_cps.append(pltpu.async_copy(ub_hbm.at[idx_u.at[k]], bu.at[sl], semb))
        bias_cps.append(pltpu.async_copy(ib_hbm.at[idx_v.at[k]], bv.at[sl], semb))

    lane = lax.iota(jnp.int32, L)

    for p in range(BPW // HALF):
        def fire(g, _, p=p):
            gg = p * (HALF // L) + g
            tu = idx_u[gg // (CHUNK // L), pl.ds((gg % (CHUNK // L)) * L, L)]
            tv = idx_v[gg // (CHUNK // L), pl.ds((gg % (CHUNK // L)) * L, L)]
            tu_t = lax.shift_right_logical(tu, 3)
            tu_s = jnp.bitwise_and(tu, 7)
            tv_t = lax.shift_right_logical(tv, 3)
            tv_s = jnp.bitwise_and(tv, 7)
            for j in range(L):
                lr = g * L + j
                pltpu.make_async_copy(
                    ue3.at[tu_t[j], tu_s[j]], U.at[lr], sem).start()
                pltpu.make_async_copy(
                    ie3.at[tv_t[j], tv_s[j]], V.at[lr], sem).start()
            return 0

        lax.fori_loop(0, HALF // L, fire, 0)

        def drain(j, _):
            pltpu.make_async_copy(ue3.at[0], U.at[pl.ds(j * 8, 8)], sem).wait()
            pltpu.make_async_copy(ie3.at[0], V.at[pl.ds(j * 8, 8)], sem).wait()
            return 0

        lax.fori_loop(0, HALF // 8, drain, 0)
        if p == 0:
            for c in bias_cps:
                c.wait()

        def group(g, _, p=p):
            grp = jnp.zeros((L,), jnp.float32)
            for r_off in range(L):
                lr = g * L + r_off
                acc = U[lr, pl.ds(0, L)] * V[lr, pl.ds(0, L)]
                for c in range(1, D // L):
                    acc = acc + U[lr, pl.ds(c * L, L)] * V[lr, pl.ds(c * L, L)]
                grp = jnp.where(lane == r_off, jnp.sum(acc), grp)
            sl = pl.ds(p * HALF + g * L, L)
            x = grp + bu[sl] + bv[sl]
            outb[sl] = 1.0 / (1.0 + jnp.exp(-x))
            return 0

        lax.fori_loop(0, HALF // L, group, 0)

    pltpu.sync_copy(outb, out_hbm.at[pl.ds(base, BPW)])


@jax.jit
def _mf_sc(u3, v3, ue3, ie3, ub, ib):
    mesh = plsc.VectorSubcoreMesh(core_axis_name="c", subcore_axis_name="s",
                                  num_cores=NC, num_subcores=NS)
    return pl.kernel(
        _body,
        out_type=jax.ShapeDtypeStruct((B,), jnp.float32),
        mesh=mesh,
        compiler_params=pltpu.CompilerParams(needs_layout_passes=False,
                                             use_tc_tiling_on_sc=True),
        scratch_types=[
            pltpu.VMEM((NCH, CHUNK), jnp.int32),
            pltpu.VMEM((NCH, CHUNK), jnp.int32),
            pltpu.VMEM((HALF, D), jnp.float32),
            pltpu.VMEM((HALF, D), jnp.float32),
            pltpu.VMEM((BPW,), jnp.float32),
            pltpu.VMEM((BPW,), jnp.float32),
            pltpu.VMEM((BPW,), jnp.float32),
            pltpu.SemaphoreType.DMA,
            pltpu.SemaphoreType.DMA,
        ],
    )(u3, v3, ue3, ie3, ub, ib)


def kernel(u, v, user_emb, item_emb, user_bias, item_bias):
    u3 = u.astype(jnp.int32).reshape(NW, NCH, CHUNK)
    v3 = v.astype(jnp.int32).reshape(NW, NCH, CHUNK)
    ue3 = user_emb.reshape(12500, 8, D)
    ie3 = item_emb.reshape(12500, 8, D)
    ub = user_bias.reshape(-1)
    ib = item_bias.reshape(-1)
    return _mf_sc(u3, v3, ue3, ie3, ub, ib)
